# baseline (speedup 1.0000x reference)
import functools

import jax
import jax.numpy as jnp
from jax import lax
from jax.experimental import pallas as pl
from jax.experimental.pallas import tpu as pltpu
from jax.experimental.pallas import tpu_sc as plsc

N = 16384
F = 128
E = 8
D = 4096

R = 256
NPAD = N + E * R
NB = NPAD // R
HALF = N // 2
L0 = HALF // R + E
L1 = HALF // R + 2 * E
NW = 32
PW = NPAD // NW
GCH = 96
NG = PW // GCH
PWH = HALF // NW
SCH = 8
NSCH = PWH // SCH

_MESH = dict(core_axis_name="c", subcore_axis_name="s")


def _sc_gather(feat, perm3):
    @functools.partial(
        pl.kernel,
        mesh=plsc.VectorSubcoreMesh(**_MESH),
        out_type=jax.ShapeDtypeStruct((NPAD, F), jnp.float32),
        scratch_types=[
            pltpu.VMEM((NG, GCH), jnp.int32),
            pltpu.VMEM((PW, F), jnp.float32),
            pltpu.SemaphoreType.DMA,
        ],
    )
    def gk(feat_hbm, idx_hbm, out_hbm, idx_v, rows_v, sem):
        wid = lax.axis_index("s") * 2 + lax.axis_index("c")
        pltpu.sync_copy(idx_hbm.at[wid], idx_v)
        copies = [
            pltpu.async_copy(
                feat_hbm.at[idx_v.at[k]], rows_v.at[pl.ds(k * GCH, GCH)], sem
            )
            for k in range(NG)
        ]
        for c in copies:
            c.wait()
        pltpu.sync_copy(rows_v, out_hbm.at[pl.ds(wid * PW, PW)])

    return gk(feat, perm3)


def _unpermute_body(row_base):
    def uk(y_hbm, idx_hbm, out_hbm, idx_v, buf_v, sem_in, sem_out):
        wid = lax.axis_index("s") * 2 + lax.axis_index("c")
        base = row_base + wid * PWH
        pltpu.sync_copy(idx_hbm.at[wid], idx_v)
        reads = [None] * NSCH
        writes = [None] * NSCH
        for c in range(3):
            reads[c] = pltpu.async_copy(
                y_hbm.at[idx_v.at[c]], buf_v.at[c], sem_in
            )
        for c in range(NSCH):
            b = c % 3
            reads[c].wait()
            writes[c] = pltpu.async_copy(
                buf_v.at[b], out_hbm.at[pl.ds(base + c * SCH, SCH)], sem_out
            )
            if c >= 1 and c + 2 < NSCH:
                writes[c - 1].wait()
                reads[c + 2] = pltpu.async_copy(
                    y_hbm.at[idx_v.at[c + 2]], buf_v.at[(c + 2) % 3], sem_in
                )
        writes[NSCH - 3].wait()
        writes[NSCH - 2].wait()
        writes[NSCH - 1].wait()

    return uk


_UNP_SCRATCH = [
    pltpu.VMEM((NSCH, SCH), jnp.int32),
    pltpu.VMEM((3, SCH, D), jnp.float32),
    pltpu.SemaphoreType.DMA,
    pltpu.SemaphoreType.DMA,
]


def _sc_unpermute_first(y, idx3):
    k = pl.kernel(
        _unpermute_body(0),
        mesh=plsc.VectorSubcoreMesh(**_MESH),
        out_type=jax.ShapeDtypeStruct((N, D), jnp.float32),
        scratch_types=_UNP_SCRATCH,
    )
    return k(y, idx3)


def _sc_unpermute_second(y, idx3, out_ref):
    k = pl.kernel(
        _unpermute_body(HALF),
        mesh=plsc.VectorSubcoreMesh(**_MESH),
        out_type=(),
        scratch_types=_UNP_SCRATCH,
    )
    k(y, idx3, out_ref)


def _tc_matmul_half(x, W16, bias3, ordh, be, bfd, lh):
    def body(ord_ref, be_ref, bfd_ref, x_ref, w_ref, b_ref, o_ref):
        i = pl.program_id(0)
        e = be_ref[ord_ref[i]]
        end = jnp.minimum(bfd_ref[e], F)
        colmask = (lax.broadcasted_iota(jnp.int32, (1, F), 1) < end).astype(
            jnp.float32
        )
        xm = (x_ref[...] * colmask).astype(jnp.bfloat16)
        acc = jnp.dot(xm, w_ref[0], preferred_element_type=jnp.float32)
        o_ref[...] = jnp.tanh(acc + b_ref[0])

    grid_spec = pltpu.PrefetchScalarGridSpec(
        num_scalar_prefetch=3,
        grid=(lh,),
        in_specs=[
            pl.BlockSpec((R, F), lambda i, o, b, f: (o[i], 0)),
            pl.BlockSpec((1, F, D), lambda i, o, b, f: (b[o[i]], 0, 0)),
            pl.BlockSpec((1, 1, D), lambda i, o, b, f: (b[o[i]], 0, 0)),
        ],
        out_specs=pl.BlockSpec((R, D), lambda i, o, b, f: (i, 0)),
    )
    return pl.pallas_call(
        body,
        grid_spec=grid_spec,
        out_shape=jax.ShapeDtypeStruct((lh * R, D), jnp.float32),
        compiler_params=pltpu.CompilerParams(dimension_semantics=("arbitrary",)),
    )(ordh, be, bfd, x, W16, bias3)


def _route(bus_type):
    bt = bus_type.astype(jnp.int32)
    counts = jnp.sum(
        (bt[:, None] == jnp.arange(E, dtype=jnp.int32)[None, :]).astype(
            jnp.int32
        ),
        axis=0,
    )
    order = jnp.argsort(bt).astype(jnp.int32)
    cum_excl = jnp.cumsum(counts) - counts
    padded = ((counts + R - 1) // R) * R
    ends_p = jnp.cumsum(padded)
    starts_p = ends_p - padded
    bstart = jnp.arange(NB, dtype=jnp.int32) * R
    bg = jnp.minimum(
        jnp.sum((bstart[:, None] >= ends_p[None, :]).astype(jnp.int32), axis=1),
        E - 1,
    )
    g = jnp.broadcast_to(bg[:, None], (NB, R)).reshape(NPAD)
    off = jnp.arange(NPAD, dtype=jnp.int32) - starts_p[g]
    src = jnp.clip(cum_excl[g] + jnp.minimum(off, counts[g] - 1), 0, N - 1)
    perm = order[src]
    first_tok = perm[::R]
    last_tok = perm[R - 1 :: R]
    block_expert = bt[first_tok]
    valid = bstart < ends_p[E - 1]
    m0 = valid & (first_tok < HALF)
    m1 = valid & (last_tok >= HALF)
    ord0_raw = jnp.nonzero(m0, size=L0, fill_value=NB)[0].astype(jnp.int32)
    ord1_raw = jnp.nonzero(m1, size=L1, fill_value=NB)[0].astype(jnp.int32)
    blkpos0 = (
        jnp.zeros((NB + 1,), jnp.int32)
        .at[ord0_raw]
        .set(jnp.arange(L0, dtype=jnp.int32))
    )
    blkpos1 = (
        jnp.zeros((NB + 1,), jnp.int32)
        .at[ord1_raw]
        .set(jnp.arange(L1, dtype=jnp.int32))
    )
    ord0 = jnp.minimum(ord0_raw, NB - 1)
    ord1 = jnp.minimum(ord1_raw, NB - 1)
    inv = jnp.zeros((N,), jnp.int32).at[order].set(
        jnp.arange(N, dtype=jnp.int32)
    )
    pos = starts_p[bt] + inv - cum_excl[bt]
    pb, po = pos // R, pos % R
    yrow = (
        jnp.where(jnp.arange(N, dtype=jnp.int32) < HALF, blkpos0[pb], blkpos1[pb])
        * R
        + po
    )
    return perm, block_expert, ord0, ord1, yrow


def kernel(feat, bus_type, bus_feature_dims, W, bias):
    perm, block_expert, ord0, ord1, yrow = _route(bus_type)
    gathered = _sc_gather(feat, perm.reshape(NW, NG, GCH))
    bfd = bus_feature_dims.astype(jnp.int32)
    W16 = W.astype(jnp.bfloat16)
    bias3 = bias.reshape(E, 1, D)
    y0 = _tc_matmul_half(gathered, W16, bias3, ord0, block_expert, bfd, L0)
    y1 = _tc_matmul_half(gathered, W16, bias3, ord1, block_expert, bfd, L1)
    out0 = _sc_unpermute_first(y0, yrow[:HALF].reshape(NW, NSCH, SCH))
    out_ref = jax.new_ref(out0)
    _sc_unpermute_second(y1, yrow[HALF:].reshape(NW, NSCH, SCH), out_ref)
    return out_ref[...]


# speedup vs baseline: 1.5108x; 1.5108x over previous
import functools

import jax
import jax.numpy as jnp
from jax import lax
from jax.experimental import pallas as pl
from jax.experimental.pallas import tpu as pltpu
from jax.experimental.pallas import tpu_sc as plsc

N = 16384
F = 128
E = 8
D = 4096

R = 256
NPAD = N + E * R
NB = NPAD // R
HALF = N // 2
L0 = HALF // R + E
L1 = HALF // R + 2 * E
NW = 32
PW = NPAD // NW
GCH = 96
NG = PW // GCH
PWH = HALF // NW
SCH = 8
NSCH = PWH // SCH

_MESH = dict(core_axis_name="c", subcore_axis_name="s")


def _sc_gather(feat, perm3):
    @functools.partial(
        pl.kernel,
        mesh=plsc.VectorSubcoreMesh(**_MESH),
        out_type=jax.ShapeDtypeStruct((NPAD, F), jnp.float32),
        scratch_types=[
            pltpu.VMEM((NG, GCH), jnp.int32),
            pltpu.VMEM((PW, F), jnp.float32),
            pltpu.SemaphoreType.DMA,
        ],
    )
    def gk(feat_hbm, idx_hbm, out_hbm, idx_v, rows_v, sem):
        wid = lax.axis_index("s") * 2 + lax.axis_index("c")
        pltpu.sync_copy(idx_hbm.at[wid], idx_v)
        copies = [
            pltpu.async_copy(
                feat_hbm.at[idx_v.at[k]], rows_v.at[pl.ds(k * GCH, GCH)], sem
            )
            for k in range(NG)
        ]
        for c in copies:
            c.wait()
        pltpu.sync_copy(rows_v, out_hbm.at[pl.ds(wid * PW, PW)])

    return gk(feat, perm3)


def _unpermute_body(row_base):
    def uk(y_hbm, idx_hbm, out_hbm, idx_v, buf_v, sem_in, sem_out):
        wid = lax.axis_index("s") * 2 + lax.axis_index("c")
        base = row_base + wid * PWH
        pltpu.sync_copy(idx_hbm.at[wid], idx_v)
        reads = [None] * NSCH
        writes = [None] * NSCH
        for c in range(3):
            reads[c] = pltpu.async_copy(
                y_hbm.at[idx_v.at[c]], buf_v.at[c], sem_in
            )
        for c in range(NSCH):
            b = c % 3
            reads[c].wait()
            writes[c] = pltpu.async_copy(
                buf_v.at[b], out_hbm.at[pl.ds(base + c * SCH, SCH)], sem_out
            )
            if c >= 1 and c + 2 < NSCH:
                writes[c - 1].wait()
                reads[c + 2] = pltpu.async_copy(
                    y_hbm.at[idx_v.at[c + 2]], buf_v.at[(c + 2) % 3], sem_in
                )
        writes[NSCH - 3].wait()
        writes[NSCH - 2].wait()
        writes[NSCH - 1].wait()

    return uk


_UNP_SCRATCH = [
    pltpu.VMEM((NSCH, SCH), jnp.int32),
    pltpu.VMEM((3, SCH, D), jnp.float32),
    pltpu.SemaphoreType.DMA,
    pltpu.SemaphoreType.DMA,
]


def _sc_unpermute_first(y, idx3):
    k = pl.kernel(
        _unpermute_body(0),
        mesh=plsc.VectorSubcoreMesh(**_MESH),
        out_type=jax.ShapeDtypeStruct((N, D), jnp.float32),
        scratch_types=_UNP_SCRATCH,
    )
    return k(y, idx3)


def _sc_unpermute_second(y, idx3, out_ref):
    k = pl.kernel(
        _unpermute_body(HALF),
        mesh=plsc.VectorSubcoreMesh(**_MESH),
        out_type=(),
        scratch_types=_UNP_SCRATCH,
    )
    k(y, idx3, out_ref)


def _tc_matmul_half(x, W16, bias3, ordh, be, bfd, lh):
    def body(ord_ref, be_ref, bfd_ref, x_ref, w_ref, b_ref, o_ref):
        i = pl.program_id(0)
        e = be_ref[ord_ref[i]]
        end = jnp.minimum(bfd_ref[e], F)
        colmask = (lax.broadcasted_iota(jnp.int32, (1, F), 1) < end).astype(
            jnp.float32
        )
        xm = (x_ref[...] * colmask).astype(jnp.bfloat16)
        acc = jnp.dot(
            xm, w_ref[0].astype(jnp.bfloat16), preferred_element_type=jnp.float32
        )
        o_ref[...] = jnp.tanh(acc + b_ref[0])

    grid_spec = pltpu.PrefetchScalarGridSpec(
        num_scalar_prefetch=3,
        grid=(lh,),
        in_specs=[
            pl.BlockSpec((R, F), lambda i, o, b, f: (o[i], 0)),
            pl.BlockSpec((1, F, D), lambda i, o, b, f: (b[o[i]], 0, 0)),
            pl.BlockSpec((1, 1, D), lambda i, o, b, f: (b[o[i]], 0, 0)),
        ],
        out_specs=pl.BlockSpec((R, D), lambda i, o, b, f: (i, 0)),
    )
    return pl.pallas_call(
        body,
        grid_spec=grid_spec,
        out_shape=jax.ShapeDtypeStruct((lh * R, D), jnp.float32),
        compiler_params=pltpu.CompilerParams(dimension_semantics=("arbitrary",)),
    )(ordh, be, bfd, x, W16, bias3)


def _route(bus_type):
    bt = bus_type.astype(jnp.int32)
    oh = (bt[:, None] == jnp.arange(E, dtype=jnp.int32)[None, :]).astype(
        jnp.int32
    )
    counts = jnp.sum(oh, axis=0)
    r0 = jnp.sum(oh[:HALF], axis=0)
    rank = jnp.take_along_axis(
        jnp.cumsum(oh, axis=0) - oh, bt[:, None], axis=1
    )[:, 0]
    order = jnp.argsort(bt).astype(jnp.int32)
    cum_excl = jnp.cumsum(counts) - counts
    padded = ((counts + R - 1) // R) * R
    ends_p = jnp.cumsum(padded)
    starts_p = ends_p - padded
    bstart = jnp.arange(NB, dtype=jnp.int32) * R
    bg = jnp.minimum(
        jnp.sum((bstart[:, None] >= ends_p[None, :]).astype(jnp.int32), axis=1),
        E - 1,
    )
    g = jnp.broadcast_to(bg[:, None], (NB, R)).reshape(NPAD)
    off = jnp.arange(NPAD, dtype=jnp.int32) - starts_p[g]
    src = jnp.clip(cum_excl[g] + jnp.minimum(off, counts[g] - 1), 0, N - 1)
    perm = order[src]
    first_tok = perm[::R]
    last_tok = perm[R - 1 :: R]
    block_expert = bt[first_tok]
    valid = bstart < ends_p[E - 1]
    m0 = valid & (first_tok < HALF)
    m1 = valid & (last_tok >= HALF)
    ord0 = jnp.minimum(
        jnp.nonzero(m0, size=L0, fill_value=NB)[0].astype(jnp.int32), NB - 1
    )
    ord1 = jnp.minimum(
        jnp.nonzero(m1, size=L1, fill_value=NB)[0].astype(jnp.int32), NB - 1
    )
    n0 = (r0 + R - 1) // R
    q0 = jnp.cumsum(n0) - n0
    n1 = jnp.where(r0 == counts, 0, (counts + R - 1) // R - r0 // R)
    q1 = jnp.cumsum(n1) - n1
    idx0 = (q0[bt] * R + rank)[:HALF]
    idx1 = ((q1[bt] + rank // R - r0[bt] // R) * R + rank % R)[HALF:]
    return perm, block_expert, ord0, ord1, idx0, idx1


def kernel(feat, bus_type, bus_feature_dims, W, bias):
    perm, block_expert, ord0, ord1, idx0, idx1 = _route(bus_type)
    gathered = _sc_gather(feat, perm.reshape(NW, NG, GCH))
    bfd = bus_feature_dims.astype(jnp.int32)
    bias3 = bias.reshape(E, 1, D)
    y0 = _tc_matmul_half(gathered, W, bias3, ord0, block_expert, bfd, L0)
    y1 = _tc_matmul_half(gathered, W, bias3, ord1, block_expert, bfd, L1)
    out0 = _sc_unpermute_first(y0, idx0.reshape(NW, NSCH, SCH))
    out_ref = jax.new_ref(out0)
    _sc_unpermute_second(y1, idx1.reshape(NW, NSCH, SCH), out_ref)
    return out_ref[...]


# speedup vs baseline: 1.6576x; 1.0972x over previous
import functools

import jax
import jax.numpy as jnp
from jax import lax
from jax.experimental import pallas as pl
from jax.experimental.pallas import tpu as pltpu
from jax.experimental.pallas import tpu_sc as plsc

N = 16384
F = 128
E = 8
D = 4096

R = 256
NPAD = N + E * R
NB = NPAD // R
NW = 32
PW = NPAD // NW
GCH = 96
NG = PW // GCH
PW2 = N // NW
SCH = 8
NSC = PW2 // SCH

_MESH = dict(core_axis_name="c", subcore_axis_name="s")


def _sc_gather(feat, perm3):
    @functools.partial(
        pl.kernel,
        mesh=plsc.VectorSubcoreMesh(**_MESH),
        out_type=jax.ShapeDtypeStruct((NPAD, F), jnp.float32),
        scratch_types=[
            pltpu.VMEM((NG, GCH), jnp.int32),
            pltpu.VMEM((PW, F), jnp.float32),
            pltpu.SemaphoreType.DMA,
        ],
    )
    def gk(feat_hbm, idx_hbm, out_hbm, idx_v, rows_v, sem):
        wid = lax.axis_index("s") * 2 + lax.axis_index("c")
        pltpu.sync_copy(idx_hbm.at[wid], idx_v)
        copies = [
            pltpu.async_copy(
                feat_hbm.at[idx_v.at[k]], rows_v.at[pl.ds(k * GCH, GCH)], sem
            )
            for k in range(NG)
        ]
        for c in copies:
            c.wait()
        pltpu.sync_copy(rows_v, out_hbm.at[pl.ds(wid * PW, PW)])

    return gk(feat, perm3)


def _sc_unpermute(y, pos3):
    @functools.partial(
        pl.kernel,
        mesh=plsc.VectorSubcoreMesh(**_MESH),
        out_type=jax.ShapeDtypeStruct((N, D), jnp.float32),
        scratch_types=[
            pltpu.VMEM((NSC, SCH), jnp.int32),
            pltpu.VMEM((3, SCH, D), jnp.float32),
            pltpu.SemaphoreType.DMA,
            pltpu.SemaphoreType.DMA,
        ],
    )
    def uk(y_hbm, idx_hbm, out_hbm, idx_v, buf_v, sem_in, sem_out):
        wid = lax.axis_index("s") * 2 + lax.axis_index("c")
        base = wid * PW2
        pltpu.sync_copy(idx_hbm.at[wid], idx_v)
        reads = [None] * NSC
        writes = [None] * NSC
        for c in range(3):
            reads[c] = pltpu.async_copy(
                y_hbm.at[idx_v.at[c]], buf_v.at[c], sem_in
            )
        for c in range(NSC):
            b = c % 3
            reads[c].wait()
            writes[c] = pltpu.async_copy(
                buf_v.at[b], out_hbm.at[pl.ds(base + c * SCH, SCH)], sem_out
            )
            if c >= 1 and c + 2 < NSC:
                writes[c - 1].wait()
                reads[c + 2] = pltpu.async_copy(
                    y_hbm.at[idx_v.at[c + 2]], buf_v.at[(c + 2) % 3], sem_in
                )
        writes[NSC - 3].wait()
        writes[NSC - 2].wait()
        writes[NSC - 1].wait()

    return uk(y, pos3)


def _tc_matmul(x, W, bias3, block_expert, bfd):
    def body(be_ref, bfd_ref, x_ref, w_ref, b_ref, o_ref):
        i = pl.program_id(0)
        e = be_ref[i]
        end = jnp.minimum(bfd_ref[e], F)
        colmask = (lax.broadcasted_iota(jnp.int32, (1, F), 1) < end).astype(
            jnp.float32
        )
        xm = (x_ref[...] * colmask).astype(jnp.bfloat16)
        acc = jnp.dot(
            xm, w_ref[0].astype(jnp.bfloat16), preferred_element_type=jnp.float32
        )
        o_ref[...] = jnp.tanh(acc + b_ref[0])

    grid_spec = pltpu.PrefetchScalarGridSpec(
        num_scalar_prefetch=2,
        grid=(NB,),
        in_specs=[
            pl.BlockSpec((R, F), lambda i, be, bfd: (i, 0)),
            pl.BlockSpec((1, F, D), lambda i, be, bfd: (be[i], 0, 0)),
            pl.BlockSpec((1, 1, D), lambda i, be, bfd: (be[i], 0, 0)),
        ],
        out_specs=pl.BlockSpec((R, D), lambda i, be, bfd: (i, 0)),
    )
    return pl.pallas_call(
        body,
        grid_spec=grid_spec,
        out_shape=jax.ShapeDtypeStruct((NPAD, D), jnp.float32),
        compiler_params=pltpu.CompilerParams(dimension_semantics=("arbitrary",)),
    )(block_expert, bfd, x, W, bias3)


def _route(bus_type):
    bt = bus_type.astype(jnp.int32)
    oh = (bt[:, None] == jnp.arange(E, dtype=jnp.int32)[None, :]).astype(
        jnp.int32
    )
    counts = jnp.sum(oh, axis=0)
    rank = jnp.take_along_axis(
        jnp.cumsum(oh, axis=0) - oh, bt[:, None], axis=1
    )[:, 0]
    order = jnp.argsort(bt).astype(jnp.int32)
    cum_excl = jnp.cumsum(counts) - counts
    padded = ((counts + R - 1) // R) * R
    ends_p = jnp.cumsum(padded)
    starts_p = ends_p - padded
    bstart = jnp.arange(NB, dtype=jnp.int32) * R
    bg = jnp.minimum(
        jnp.sum((bstart[:, None] >= ends_p[None, :]).astype(jnp.int32), axis=1),
        E - 1,
    )
    g = jnp.broadcast_to(bg[:, None], (NB, R)).reshape(NPAD)
    off = jnp.arange(NPAD, dtype=jnp.int32) - starts_p[g]
    src = jnp.clip(cum_excl[g] + jnp.minimum(off, counts[g] - 1), 0, N - 1)
    perm = order[src]
    block_expert = bt[perm[::R]]
    pos = starts_p[bt] + rank
    return perm, pos, block_expert


def kernel(feat, bus_type, bus_feature_dims, W, bias):
    perm, pos, block_expert = _route(bus_type)
    gathered = _sc_gather(feat, perm.reshape(NW, NG, GCH))
    y = _tc_matmul(
        gathered, W, bias.reshape(E, 1, D), block_expert,
        bus_feature_dims.astype(jnp.int32),
    )
    return _sc_unpermute(y, pos.reshape(NW, NSC, SCH))


# speedup vs baseline: 1.7006x; 1.0259x over previous
import functools

import jax
import jax.numpy as jnp
from jax import lax
from jax.experimental import pallas as pl
from jax.experimental.pallas import tpu as pltpu
from jax.experimental.pallas import tpu_sc as plsc

N = 16384
F = 128
E = 8
D = 4096

R = 256
NPAD = N + E * R
NB = NPAD // R
NW = 32
PW = NPAD // NW
GCH = 96
NG = PW // GCH
PW2 = N // NW
SCH = 8
NSC = PW2 // SCH

_MESH = dict(core_axis_name="c", subcore_axis_name="s")


def _sc_gather(feat, perm3):
    @functools.partial(
        pl.kernel,
        mesh=plsc.VectorSubcoreMesh(**_MESH),
        out_type=jax.ShapeDtypeStruct((NPAD, F), jnp.float32),
        scratch_types=[
            pltpu.VMEM((NG, GCH), jnp.int32),
            pltpu.VMEM((PW, F), jnp.float32),
            pltpu.SemaphoreType.DMA,
        ],
    )
    def gk(feat_hbm, idx_hbm, out_hbm, idx_v, rows_v, sem):
        wid = lax.axis_index("s") * 2 + lax.axis_index("c")
        pltpu.sync_copy(idx_hbm.at[wid], idx_v)
        copies = [
            pltpu.async_copy(
                feat_hbm.at[idx_v.at[k]], rows_v.at[pl.ds(k * GCH, GCH)], sem
            )
            for k in range(NG)
        ]
        for c in copies:
            c.wait()
        pltpu.sync_copy(rows_v, out_hbm.at[pl.ds(wid * PW, PW)])

    return gk(feat, perm3)


def _sc_unpermute(y, pos3):
    @functools.partial(
        pl.kernel,
        mesh=plsc.VectorSubcoreMesh(**_MESH),
        out_type=jax.ShapeDtypeStruct((N, D), jnp.float32),
        scratch_types=[
            pltpu.VMEM((NSC, SCH), jnp.int32),
            pltpu.VMEM((3, SCH, D), jnp.float32),
            pltpu.SemaphoreType.DMA,
            pltpu.SemaphoreType.DMA,
        ],
    )
    def uk(y_hbm, idx_hbm, out_hbm, idx_v, buf_v, sem_in, sem_out):
        wid = lax.axis_index("s") * 2 + lax.axis_index("c")
        base = wid * PW2
        pltpu.sync_copy(idx_hbm.at[wid], idx_v)
        reads = [None] * NSC
        writes = [None] * NSC
        for c in range(3):
            reads[c] = pltpu.async_copy(
                y_hbm.at[idx_v.at[c]], buf_v.at[c], sem_in
            )
        for c in range(NSC):
            b = c % 3
            reads[c].wait()
            writes[c] = pltpu.async_copy(
                buf_v.at[b], out_hbm.at[pl.ds(base + c * SCH, SCH)], sem_out
            )
            if c >= 1 and c + 2 < NSC:
                writes[c - 1].wait()
                reads[c + 2] = pltpu.async_copy(
                    y_hbm.at[idx_v.at[c + 2]], buf_v.at[(c + 2) % 3], sem_in
                )
        writes[NSC - 3].wait()
        writes[NSC - 2].wait()
        writes[NSC - 1].wait()

    return uk(y, pos3)


def _tc_matmul(x, W, bias3, block_expert, bfd):
    def body(be_ref, bfd_ref, x_ref, w_ref, b_ref, o_ref):
        i = pl.program_id(0)
        e = be_ref[i]
        end = jnp.minimum(bfd_ref[e], F)
        colmask = (lax.broadcasted_iota(jnp.int32, (1, F), 1) < end).astype(
            jnp.float32
        )
        xm = (x_ref[...] * colmask).astype(jnp.bfloat16)
        acc = jnp.dot(xm, w_ref[0], preferred_element_type=jnp.float32)
        o_ref[...] = jnp.tanh(acc + b_ref[0])

    grid_spec = pltpu.PrefetchScalarGridSpec(
        num_scalar_prefetch=2,
        grid=(NB,),
        in_specs=[
            pl.BlockSpec((R, F), lambda i, be, bfd: (i, 0)),
            pl.BlockSpec((1, F, D), lambda i, be, bfd: (be[i], 0, 0)),
            pl.BlockSpec((1, 1, D), lambda i, be, bfd: (be[i], 0, 0)),
        ],
        out_specs=pl.BlockSpec((R, D), lambda i, be, bfd: (i, 0)),
    )
    return pl.pallas_call(
        body,
        grid_spec=grid_spec,
        out_shape=jax.ShapeDtypeStruct((NPAD, D), jnp.float32),
        compiler_params=pltpu.CompilerParams(dimension_semantics=("arbitrary",)),
    )(block_expert, bfd, x, W, bias3)


def _route(bus_type):
    bt = bus_type.astype(jnp.int32)
    counts = jnp.sum(
        (bt[:, None] == jnp.arange(E, dtype=jnp.int32)[None, :]).astype(
            jnp.int32
        ),
        axis=0,
    )
    order = jnp.argsort(bt).astype(jnp.int32)
    cum_excl = jnp.cumsum(counts) - counts
    padded = ((counts + R - 1) // R) * R
    ends_p = jnp.cumsum(padded)
    starts_p = ends_p - padded
    bstart = jnp.arange(NB, dtype=jnp.int32) * R
    bg = jnp.minimum(
        jnp.sum((bstart[:, None] >= ends_p[None, :]).astype(jnp.int32), axis=1),
        E - 1,
    )
    g = jnp.broadcast_to(bg[:, None], (NB, R)).reshape(NPAD)
    off = jnp.arange(NPAD, dtype=jnp.int32) - starts_p[g]
    src = jnp.clip(cum_excl[g] + jnp.minimum(off, counts[g] - 1), 0, N - 1)
    perm = order[src]
    block_expert = bt[perm[::R]]
    inv = jnp.zeros((N,), jnp.int32).at[order].set(
        jnp.arange(N, dtype=jnp.int32)
    )
    pos = starts_p[bt] + inv - cum_excl[bt]
    return perm, pos, block_expert


def kernel(feat, bus_type, bus_feature_dims, W, bias):
    perm, pos, block_expert = _route(bus_type)
    gathered = _sc_gather(feat, perm.reshape(NW, NG, GCH))
    y = _tc_matmul(
        gathered, W.astype(jnp.bfloat16), bias.reshape(E, 1, D), block_expert,
        bus_feature_dims.astype(jnp.int32),
    )
    return _sc_unpermute(y, pos.reshape(NW, NSC, SCH))
